# baseline (speedup 1.0000x reference)
import jax
import jax.numpy as jnp
from jax import lax
from jax.experimental import pallas as pl
from jax.experimental.pallas import tpu as pltpu
from jax.experimental.pallas import tpu_sc as plsc

NC = 2
NS = 16
NW = NC * NS

LANES = 16

BATCH = 4096
MAXLEN = 200
EMBED = 32

BLK = BATCH // NW
LCH = 10
NCHUNK = MAXLEN // LCH


def _body(xw_hbm, tok_hbm, pos_hbm, out_hbm, idx_v, pos_v,
          buf0, buf1, tb0, tb1, g0, g1, o0, o1):
    wid = lax.axis_index("s") * NC + lax.axis_index("c")

    pltpu.sync_copy(pos_hbm, pos_v)
    pltpu.sync_copy(xw_hbm.at[wid], idx_v)

    iota = lax.iota(jnp.int32, LANES)
    idx_b = [iota + g * LANES for g in range(BLK // LANES)]

    bufs = (buf0, buf1)
    tbufs = (tb0, tb1)
    gsems = (g0, g1)
    osems = (o0, o1)

    def _fire_chunk(c, pc):
        for j in range(LCH):
            pltpu.async_copy(
                tok_hbm.at[idx_v.at[c * LCH + j]], bufs[pc].at[j], gsems[pc]
            )

    def _drain_chunk(c, pc):
        for j in range(LCH):
            pltpu.make_async_copy(
                tok_hbm.at[idx_v.at[c * LCH + j]], bufs[pc].at[j], gsems[pc]
            ).wait()

    def _fire_out(l, tp):
        for d_hi in range(4):
            pltpu.async_copy(
                tbufs[tp].at[d_hi], out_hbm.at[l, d_hi, wid], osems[tp]
            )

    def _drain_out(l, tp):
        for d_hi in range(4):
            pltpu.make_async_copy(
                tbufs[tp].at[d_hi], out_hbm.at[l, d_hi, wid], osems[tp]
            ).wait()

    def _transpose_add(pc, j, l, tp):
        j_splat = jnp.zeros((LANES,), jnp.int32) + j
        l_splat = jnp.zeros((LANES,), jnp.int32) + l
        for d in range(EMBED):
            d_splat = jnp.full((LANES,), d, jnp.int32)
            p_d = plsc.load_gather(pos_v, [l_splat, d_splat])
            for g in range(BLK // LANES):
                v = plsc.load_gather(bufs[pc], [j_splat, idx_b[g], d_splat])
                tbufs[tp][d // 8, d % 8, pl.ds(g * LANES, LANES)] = v + p_d

    _fire_chunk(0, 0)

    def _chunk_step(c, pc):
        @pl.when(c < NCHUNK - 1)
        def _():
            _fire_chunk(c + 1, 1 - pc)

        _drain_chunk(c, pc)

        def inner(k, carry):
            for tp in (0, 1):
                j = 2 * k + tp
                l = c * LCH + j
                @pl.when(l >= 2)
                def _():
                    _drain_out(l - 2, tp)

                _transpose_add(pc, j, l, tp)
                _fire_out(l, tp)
            return carry

        lax.fori_loop(0, LCH // 2, inner, 0)

    def pair_body(m, carry):
        _chunk_step(2 * m, 0)
        _chunk_step(2 * m + 1, 1)
        return carry

    lax.fori_loop(0, NCHUNK // 2, pair_body, 0)

    _drain_out(MAXLEN - 2, 0)
    _drain_out(MAXLEN - 1, 1)


@jax.jit
def kernel(x, token_table, pos_table):
    mesh = plsc.VectorSubcoreMesh(
        core_axis_name="c", subcore_axis_name="s", num_cores=NC, num_subcores=NS
    )
    xw = jnp.transpose(x.reshape(NW, BLK, MAXLEN), (0, 2, 1))
    out5 = pl.kernel(
        _body,
        out_type=jax.ShapeDtypeStruct((MAXLEN, 4, NW, 8, BLK), jnp.float32),
        mesh=mesh,
        scratch_types=[
            pltpu.VMEM((MAXLEN, BLK), jnp.int32),
            pltpu.VMEM((MAXLEN, EMBED), jnp.float32),
            pltpu.VMEM((LCH, BLK, EMBED), jnp.float32),
            pltpu.VMEM((LCH, BLK, EMBED), jnp.float32),
            pltpu.VMEM((4, 8, BLK), jnp.float32),
            pltpu.VMEM((4, 8, BLK), jnp.float32),
            pltpu.SemaphoreType.DMA,
            pltpu.SemaphoreType.DMA,
            pltpu.SemaphoreType.DMA,
            pltpu.SemaphoreType.DMA,
        ],
        compiler_params=pltpu.CompilerParams(
            use_tc_tiling_on_sc=False, needs_layout_passes=False
        ),
    )(xw, token_table, pos_table)
    return out5.transpose(2, 4, 0, 1, 3).reshape(BATCH, MAXLEN, EMBED)


# speedup vs baseline: 1.1849x; 1.1849x over previous
import jax
import jax.numpy as jnp
from jax import lax
from jax.experimental import pallas as pl
from jax.experimental.pallas import tpu as pltpu
from jax.experimental.pallas import tpu_sc as plsc

NC = 2
NS = 16
NW = NC * NS

LANES = 16

BATCH = 4096
MAXLEN = 200
EMBED = 32

BLK = BATCH // NW
LCH = 10
NCHUNK = MAXLEN // LCH


def _body(xw_hbm, tok_hbm, pos_hbm, out_hbm, idx_v, pos_v,
          buf0, buf1, tb0, tb1, g0, g1, o0, o1):
    wid = lax.axis_index("s") * NC + lax.axis_index("c")

    pltpu.sync_copy(pos_hbm, pos_v)
    pltpu.sync_copy(xw_hbm.at[wid], idx_v)

    iota = lax.iota(jnp.int32, LANES)
    idx_b = [iota + g * LANES for g in range(BLK // LANES)]

    bufs = (buf0, buf1)
    tbufs = (tb0, tb1)
    gsems = (g0, g1)
    osems = (o0, o1)

    def _fire_chunk(c, pc):
        for j in range(LCH):
            pltpu.async_copy(
                tok_hbm.at[idx_v.at[c * LCH + j]], bufs[pc].at[j], gsems[pc]
            )

    def _drain_chunk(c, pc):
        for j in range(LCH):
            pltpu.make_async_copy(
                tok_hbm.at[idx_v.at[c * LCH + j]], bufs[pc].at[j], gsems[pc]
            ).wait()

    def _fire_out(l, tp):
        for d_hi in range(4):
            pltpu.async_copy(
                tbufs[tp].at[pl.ds(d_hi * 8 * BLK, 8 * BLK)],
                out_hbm.at[l, d_hi, wid],
                osems[tp],
            )

    def _drain_out(l, tp):
        for d_hi in range(4):
            pltpu.make_async_copy(
                tbufs[tp].at[pl.ds(d_hi * 8 * BLK, 8 * BLK)],
                out_hbm.at[l, d_hi, wid],
                osems[tp],
            ).wait()

    p0c = (iota // 8) * (8 * BLK) + (iota % 8) * BLK
    p1c = p0c + 2 * 8 * BLK

    def _transpose_add(pc, j, l, tp):
        p0 = pos_v[l, pl.ds(0, LANES)]
        p1 = pos_v[l, pl.ds(LANES, LANES)]
        for b in range(BLK):
            v0 = bufs[pc][j, b, pl.ds(0, LANES)]
            v1 = bufs[pc][j, b, pl.ds(LANES, LANES)]
            plsc.store_scatter(tbufs[tp], [p0c + b], v0 + p0)
            plsc.store_scatter(tbufs[tp], [p1c + b], v1 + p1)

    _fire_chunk(0, 0)

    def _chunk_step(c, pc):
        @pl.when(c < NCHUNK - 1)
        def _():
            _fire_chunk(c + 1, 1 - pc)

        _drain_chunk(c, pc)

        def inner(k, carry):
            for tp in (0, 1):
                j = 2 * k + tp
                l = c * LCH + j
                @pl.when(l >= 2)
                def _():
                    _drain_out(l - 2, tp)

                _transpose_add(pc, j, l, tp)
                _fire_out(l, tp)
            return carry

        lax.fori_loop(0, LCH // 2, inner, 0)

    def pair_body(m, carry):
        _chunk_step(2 * m, 0)
        _chunk_step(2 * m + 1, 1)
        return carry

    lax.fori_loop(0, NCHUNK // 2, pair_body, 0)

    _drain_out(MAXLEN - 2, 0)
    _drain_out(MAXLEN - 1, 1)


@jax.jit
def kernel(x, token_table, pos_table):
    mesh = plsc.VectorSubcoreMesh(
        core_axis_name="c", subcore_axis_name="s", num_cores=NC, num_subcores=NS
    )
    xw = jnp.transpose(x.reshape(NW, BLK, MAXLEN), (0, 2, 1))
    out5 = pl.kernel(
        _body,
        out_type=jax.ShapeDtypeStruct((MAXLEN, 4, NW, 8 * BLK), jnp.float32),
        mesh=mesh,
        scratch_types=[
            pltpu.VMEM((MAXLEN, BLK), jnp.int32),
            pltpu.VMEM((MAXLEN, EMBED), jnp.float32),
            pltpu.VMEM((LCH, BLK, EMBED), jnp.float32),
            pltpu.VMEM((LCH, BLK, EMBED), jnp.float32),
            pltpu.VMEM((4 * 8 * BLK,), jnp.float32),
            pltpu.VMEM((4 * 8 * BLK,), jnp.float32),
            pltpu.SemaphoreType.DMA,
            pltpu.SemaphoreType.DMA,
            pltpu.SemaphoreType.DMA,
            pltpu.SemaphoreType.DMA,
        ],
        compiler_params=pltpu.CompilerParams(
            use_tc_tiling_on_sc=False, needs_layout_passes=False
        ),
    )(xw, token_table, pos_table)
    out5 = out5.reshape(MAXLEN, 4, NW, 8, BLK)
    return out5.transpose(2, 4, 0, 1, 3).reshape(BATCH, MAXLEN, EMBED)


# speedup vs baseline: 1.5987x; 1.3493x over previous
import jax
import jax.numpy as jnp
from jax import lax
from jax.experimental import pallas as pl
from jax.experimental.pallas import tpu as pltpu
from jax.experimental.pallas import tpu_sc as plsc

NC = 2
NS = 16
NW = NC * NS

LANES = 16

BATCH = 4096
MAXLEN = 200
EMBED = 32

BLK = BATCH // NW
LCH = 10
NCHUNK = MAXLEN // LCH


def _body(xw_hbm, tok_hbm, pos_hbm, out_hbm, idx_v, pos_v,
          buf0, buf1, tb0, tb1, g0, g1, o0, o1):
    wid = lax.axis_index("s") * NC + lax.axis_index("c")

    pltpu.sync_copy(pos_hbm, pos_v)
    pltpu.sync_copy(xw_hbm.at[wid], idx_v)

    iota = lax.iota(jnp.int32, LANES)
    idx_b = [iota + g * LANES for g in range(BLK // LANES)]

    bufs = (buf0, buf1)
    tbufs = (tb0, tb1)
    gsems = (g0, g1)
    osems = (o0, o1)

    def _fire_chunk(c, pc):
        for j in range(LCH):
            pltpu.async_copy(
                tok_hbm.at[idx_v.at[c * LCH + j]], bufs[pc].at[j], gsems[pc]
            )

    def _drain_chunk(c, pc):
        for j in range(LCH):
            pltpu.make_async_copy(
                tok_hbm.at[idx_v.at[c * LCH + j]], bufs[pc].at[j], gsems[pc]
            ).wait()

    def _fire_out(l, tp):
        for d_hi in range(4):
            pltpu.async_copy(
                tbufs[tp].at[pl.ds(d_hi * 8 * BLK, 8 * BLK)],
                out_hbm.at[l, d_hi, wid],
                osems[tp],
            )

    def _drain_out(l, tp):
        for d_hi in range(4):
            pltpu.make_async_copy(
                tbufs[tp].at[pl.ds(d_hi * 8 * BLK, 8 * BLK)],
                out_hbm.at[l, d_hi, wid],
                osems[tp],
            ).wait()

    rotv = [(iota + k) & (LANES - 1) for k in range(LANES)]
    avec = (iota // 8) * (8 * BLK) + (iota % 8) * BLK

    def _transpose_add(pc, j, l, tp):
        j_splat = jnp.zeros((LANES,), jnp.int32) + j
        d_vecs = [iota + h * LANES for h in range(EMBED // LANES)]
        p_hs = [pos_v[l, pl.ds(h * LANES, LANES)] for h in range(EMBED // LANES)]
        a_hs = [avec + 2 * 8 * BLK * h for h in range(EMBED // LANES)]

        def bb_body(bb, carry):
            b0 = bb * LANES
            for h in range(EMBED // LANES):
                for k in range(LANES):
                    v = plsc.load_gather(
                        bufs[pc], [j_splat, rotv[k] + b0, d_vecs[h]]
                    )
                    plsc.store_scatter(
                        tbufs[tp], [a_hs[h] + b0 + rotv[k]], v + p_hs[h]
                    )
            return carry

        lax.fori_loop(0, BLK // LANES, bb_body, 0)

    _fire_chunk(0, 0)

    def _chunk_step(c, pc):
        @pl.when(c < NCHUNK - 1)
        def _():
            _fire_chunk(c + 1, 1 - pc)

        _drain_chunk(c, pc)

        def inner(k, carry):
            for tp in (0, 1):
                j = 2 * k + tp
                l = c * LCH + j
                @pl.when(l >= 2)
                def _():
                    _drain_out(l - 2, tp)

                _transpose_add(pc, j, l, tp)
                _fire_out(l, tp)
            return carry

        lax.fori_loop(0, LCH // 2, inner, 0)

    def pair_body(m, carry):
        _chunk_step(2 * m, 0)
        _chunk_step(2 * m + 1, 1)
        return carry

    lax.fori_loop(0, NCHUNK // 2, pair_body, 0)

    _drain_out(MAXLEN - 2, 0)
    _drain_out(MAXLEN - 1, 1)


@jax.jit
def kernel(x, token_table, pos_table):
    mesh = plsc.VectorSubcoreMesh(
        core_axis_name="c", subcore_axis_name="s", num_cores=NC, num_subcores=NS
    )
    xw = jnp.transpose(x.reshape(NW, BLK, MAXLEN), (0, 2, 1))
    out5 = pl.kernel(
        _body,
        out_type=jax.ShapeDtypeStruct((MAXLEN, 4, NW, 8 * BLK), jnp.float32),
        mesh=mesh,
        scratch_types=[
            pltpu.VMEM((MAXLEN, BLK), jnp.int32),
            pltpu.VMEM((MAXLEN, EMBED), jnp.float32),
            pltpu.VMEM((LCH, BLK, EMBED), jnp.float32),
            pltpu.VMEM((LCH, BLK, EMBED), jnp.float32),
            pltpu.VMEM((4 * 8 * BLK,), jnp.float32),
            pltpu.VMEM((4 * 8 * BLK,), jnp.float32),
            pltpu.SemaphoreType.DMA,
            pltpu.SemaphoreType.DMA,
            pltpu.SemaphoreType.DMA,
            pltpu.SemaphoreType.DMA,
        ],
        compiler_params=pltpu.CompilerParams(
            use_tc_tiling_on_sc=False, needs_layout_passes=False
        ),
    )(xw, token_table, pos_table)
    out5 = out5.reshape(MAXLEN, 4, NW, 8, BLK)
    return out5.transpose(2, 4, 0, 1, 3).reshape(BATCH, MAXLEN, EMBED)
